# baseline (speedup 1.0000x reference)
import jax
import jax.numpy as jnp
from jax.experimental import pallas as pl
from jax.experimental.pallas import tpu as pltpu

_BN = 8


def _aol_kernel(x_ref, w_ref, sp_ref, out_ref, scale_ref):
    @pl.when(pl.program_id(0) == 0)
    def _compute_scale():
        xf = jnp.dot(w_ref[...], sp_ref[...], preferred_element_type=jnp.float32)
        scale_ref[...] = 1.0 + jax.nn.sigmoid(xf)

    out_ref[...] = x_ref[...] * scale_ref[...][None, :, :]


def kernel(inputs, labels, cpct_r_w, conv_w, similar_prototype):
    n, c, h, w = inputs.shape
    hw = h * w
    x = inputs.reshape(n, c, hw)
    sp = similar_prototype.reshape(c, hw)

    out = pl.pallas_call(
        _aol_kernel,
        grid=(n // _BN,),
        in_specs=[
            pl.BlockSpec((_BN, c, hw), lambda i: (i, 0, 0)),
            pl.BlockSpec((c, c), lambda i: (0, 0)),
            pl.BlockSpec((c, hw), lambda i: (0, 0)),
        ],
        out_specs=pl.BlockSpec((_BN, c, hw), lambda i: (i, 0, 0)),
        out_shape=jax.ShapeDtypeStruct((n, c, hw), inputs.dtype),
        scratch_shapes=[pltpu.VMEM((c, hw), jnp.float32)],
    )(x, conv_w, sp)
    return out.reshape(n, c, h, w)
